# Initial kernel scaffold; baseline (speedup 1.0000x reference)
import jax
import jax.numpy as jnp
from jax.experimental import pallas as pl


def kernel(x, edge_index, W1, b1, W2, b2, W3, b3):
    raise NotImplementedError("write your pallas kernel here")


# speedup vs baseline: 22.7224x; 22.7224x over previous
import functools

import jax
import jax.numpy as jnp
from jax import lax
from jax.experimental import pallas as pl
from jax.experimental.pallas import tpu as pltpu
from jax.experimental.pallas import tpu_sc as plsc

N = 10000
E = 320000
IN, HID, MID = 128, 256, 64

NP = 10240
NTILES = 32
KIDX = 128
R = -(-E // (NTILES * KIDX))
EP = NTILES * KIDX * R
R2 = EP // (16 * KIDX)
RPT = NP // 16
BN = 1024


def _mesh():
    return plsc.VectorSubcoreMesh(core_axis_name="c", subcore_axis_name="s")


_SC_PARAMS = pltpu.CompilerParams(use_tc_tiling_on_sc=False)


def _make_agg(FH, CH):
    @functools.partial(
        pl.kernel,
        out_type=jax.ShapeDtypeStruct((2, NP, FH), jnp.float32),
        mesh=_mesh(),
        compiler_params=_SC_PARAMS,
        scratch_types=[
            pltpu.VMEM((R2, KIDX), jnp.int32),
            pltpu.VMEM((R2, KIDX), jnp.int32),
            pltpu.VMEM((KIDX, FH), jnp.float32),
            pltpu.VMEM((CH, FH), jnp.float32),
            pltpu.VMEM_SHARED((NP, FH), jnp.float32),
            pltpu.SemaphoreType.DMA,
        ],
    )
    def agg(table, srcw, dstw, out, src_v, dst_v, rows_v, obuf, acc, sem):
        c = lax.axis_index("c")
        s = lax.axis_index("s")
        pltpu.sync_copy(srcw.at[c, s], src_v)
        pltpu.sync_copy(dstw.at[s], dst_v)

        zv = jnp.zeros((16,), jnp.float32)

        def zrow(i, carry):
            for j in range(FH // 16):
                obuf[i, pl.ds(j * 16, 16)] = zv
            return carry

        lax.fori_loop(0, CH, zrow, 0)
        for k in range(RPT // CH):
            pltpu.sync_copy(obuf, acc.at[pl.ds(s * RPT + k * CH, CH)])
        plsc.subcore_barrier()

        def body(r, carry):
            pltpu.async_copy(table.at[src_v.at[r]], rows_v, sem).wait()
            pltpu.sync_copy(rows_v, acc.at[dst_v.at[r]], add=True)
            return carry

        lax.fori_loop(0, R2, body, 0)
        plsc.subcore_barrier()
        for k in range(RPT // CH):
            off = s * RPT + k * CH
            pltpu.sync_copy(acc.at[pl.ds(off, CH)], obuf)
            pltpu.sync_copy(obuf, out.at[c].at[pl.ds(off, CH)])

    return agg


@functools.partial(
    pl.kernel,
    out_type=jax.ShapeDtypeStruct((2, NP), jnp.float32),
    mesh=_mesh(),
    compiler_params=_SC_PARAMS,
    scratch_types=[
        pltpu.VMEM((R, KIDX), jnp.int32),
        pltpu.VMEM((KIDX,), jnp.float32),
        pltpu.VMEM((RPT,), jnp.float32),
        pltpu.VMEM_SHARED((NP,), jnp.float32),
    ],
)
def _deg_kernel(dstw, out, dst_v, ones_v, obuf, acc):
    c = lax.axis_index("c")
    s = lax.axis_index("s")
    wid = s * 2 + c
    pltpu.sync_copy(dstw.at[wid], dst_v)
    for j in range(KIDX // 16):
        ones_v[pl.ds(j * 16, 16)] = jnp.ones((16,), jnp.float32)
    for j in range(RPT // 16):
        obuf[pl.ds(j * 16, 16)] = jnp.zeros((16,), jnp.float32)
    pltpu.sync_copy(obuf, acc.at[pl.ds(s * RPT, RPT)])
    plsc.subcore_barrier()

    def body(r, carry):
        pltpu.sync_copy(ones_v, acc.at[dst_v.at[r]], add=True)
        return carry

    lax.fori_loop(0, R, body, 0)
    plsc.subcore_barrier()
    pltpu.sync_copy(acc.at[pl.ds(s * RPT, RPT)], obuf)
    pltpu.sync_copy(obuf, out.at[c].at[pl.ds(s * RPT, RPT)])


@functools.partial(
    pl.kernel,
    out_type=jax.ShapeDtypeStruct((2, NP), jnp.float32),
    mesh=_mesh(),
    compiler_params=_SC_PARAMS,
    scratch_types=[
        pltpu.VMEM((R, KIDX), jnp.int32),
        pltpu.VMEM((R, KIDX), jnp.int32),
        pltpu.VMEM((KIDX,), jnp.float32),
        pltpu.VMEM((RPT,), jnp.float32),
        pltpu.VMEM_SHARED((NP,), jnp.float32),
        pltpu.SemaphoreType.DMA,
    ],
)
def _agg1_kernel(table, srcw, dstw, out, src_v, dst_v, rows_v, obuf, acc, sem):
    c = lax.axis_index("c")
    s = lax.axis_index("s")
    wid = s * 2 + c
    pltpu.sync_copy(srcw.at[wid], src_v)
    pltpu.sync_copy(dstw.at[wid], dst_v)
    for j in range(RPT // 16):
        obuf[pl.ds(j * 16, 16)] = jnp.zeros((16,), jnp.float32)
    pltpu.sync_copy(obuf, acc.at[pl.ds(s * RPT, RPT)])
    plsc.subcore_barrier()

    def body(r, carry):
        pltpu.async_copy(table.at[src_v.at[r]], rows_v, sem).wait()
        pltpu.sync_copy(rows_v, acc.at[dst_v.at[r]], add=True)
        return carry

    lax.fori_loop(0, R, body, 0)
    plsc.subcore_barrier()
    pltpu.sync_copy(acc.at[pl.ds(s * RPT, RPT)], obuf)
    pltpu.sync_copy(obuf, out.at[c].at[pl.ds(s * RPT, RPT)])


_agg128 = _make_agg(IN // 2, 320)
_agg64 = _make_agg(MID // 2, 640)


def _t1(degp, x_p):
    def body(dg_ref, x_ref, d_ref, xs_ref):
        dp = dg_ref[...]
        dcol = lax.rsqrt(dp[0] + dp[1] + 1.0)
        d_ref[...] = dcol
        xs = dcol * x_ref[...]
        xs_ref[...] = jnp.stack([xs[:, : IN // 2], xs[:, IN // 2:]], axis=0)

    return pl.pallas_call(
        body,
        grid=(NP // BN,),
        in_specs=[
            pl.BlockSpec((2, BN, 1), lambda i: (0, i, 0)),
            pl.BlockSpec((BN, IN), lambda i: (i, 0)),
        ],
        out_specs=[
            pl.BlockSpec((BN, 1), lambda i: (i, 0)),
            pl.BlockSpec((2, BN, IN // 2), lambda i: (0, i, 0)),
        ],
        out_shape=[
            jax.ShapeDtypeStruct((NP, 1), jnp.float32),
            jax.ShapeDtypeStruct((2, NP, IN // 2), jnp.float32),
        ],
    )(degp, x_p)


def _t2(aggp, x_p, d, W1, b1, W2):
    def body(a_ref, x_ref, d_ref, w1_ref, b1_ref, w2_ref, t2_ref):
        d = d_ref[...]
        a = jnp.concatenate([a_ref[0], a_ref[1]], axis=1)
        ax = d * a + (d * d) * x_ref[...]
        h1 = jnp.maximum(
            jnp.dot(ax, w1_ref[...], preferred_element_type=jnp.float32)
            + b1_ref[...], 0.0)
        p2 = jnp.dot(h1, w2_ref[...], preferred_element_type=jnp.float32)
        t2 = d * p2
        t2_ref[...] = jnp.stack([t2[:, : MID // 2], t2[:, MID // 2:]], axis=0)

    return pl.pallas_call(
        body,
        grid=(NP // BN,),
        in_specs=[
            pl.BlockSpec((2, BN, IN // 2), lambda i: (0, i, 0)),
            pl.BlockSpec((BN, IN), lambda i: (i, 0)),
            pl.BlockSpec((BN, 1), lambda i: (i, 0)),
            pl.BlockSpec((IN, HID), lambda i: (0, 0)),
            pl.BlockSpec((1, HID), lambda i: (0, 0)),
            pl.BlockSpec((HID, MID), lambda i: (0, 0)),
        ],
        out_specs=pl.BlockSpec((2, BN, MID // 2), lambda i: (0, i, 0)),
        out_shape=jax.ShapeDtypeStruct((2, NP, MID // 2), jnp.float32),
    )(aggp, x_p, d, W1, b1, W2)


def _t3(aggp, t2, d, b2, w3r):
    def body(a_ref, t2_ref, d_ref, b2_ref, w3_ref, t3_ref):
        d = d_ref[...]
        a = jnp.concatenate([a_ref[0] + t2_ref[0], a_ref[1] + t2_ref[1]], axis=1)
        h2 = jnp.maximum(d * a + b2_ref[...], 0.0)
        p3 = jnp.sum(h2 * w3_ref[...], axis=1, keepdims=True)
        t3_ref[...] = d * p3

    return pl.pallas_call(
        body,
        grid=(NP // BN,),
        in_specs=[
            pl.BlockSpec((2, BN, MID // 2), lambda i: (0, i, 0)),
            pl.BlockSpec((2, BN, MID // 2), lambda i: (0, i, 0)),
            pl.BlockSpec((BN, 1), lambda i: (i, 0)),
            pl.BlockSpec((1, MID), lambda i: (0, 0)),
            pl.BlockSpec((1, MID), lambda i: (0, 0)),
        ],
        out_specs=pl.BlockSpec((BN, 1), lambda i: (i, 0)),
        out_shape=jax.ShapeDtypeStruct((NP, 1), jnp.float32),
    )(aggp, t2, d, b2, w3r)


def _t4(aggp, t3, d, b3):
    def body(a_ref, t3_ref, d_ref, b3_ref, o_ref):
        o_ref[...] = d_ref[...] * (a_ref[0] + a_ref[1] + t3_ref[...]) + b3_ref[0, 0]

    return pl.pallas_call(
        body,
        out_shape=jax.ShapeDtypeStruct((NP // 128, 128), jnp.float32),
    )(aggp, t3, d, b3)


def kernel(x, edge_index, W1, b1, W2, b2, W3, b3):
    src = edge_index[0]
    dst = edge_index[1]
    pad = EP - E
    pad_idx = N + (jnp.arange(pad, dtype=jnp.int32) % (NP - N))
    src_f = jnp.concatenate([src, pad_idx])
    dst_f = jnp.concatenate([dst, pad_idx])
    src_p = src_f.reshape(NTILES, R, KIDX)
    dst_p = dst_f.reshape(NTILES, R, KIDX)
    src_pc = jnp.stack([src_f, src_f + NP]).reshape(2, 16, R2, KIDX)
    dst_pc = dst_f.reshape(16, R2, KIDX)
    x_p = jnp.pad(x, ((0, NP - N), (0, 0)))

    degp = _deg_kernel(dst_p)
    d, xs = _t1(degp.reshape(2, NP, 1), x_p)
    agg1 = _agg128(xs.reshape(2 * NP, IN // 2), src_pc, dst_pc)
    t2 = _t2(agg1, x_p, d, W1, b1.reshape(1, HID), W2)
    agg2 = _agg64(t2.reshape(2 * NP, MID // 2), src_pc, dst_pc)
    t3 = _t3(agg2, t2, d, b2.reshape(1, MID), W3.reshape(1, MID))
    agg3 = _agg1_kernel(t3.reshape(NP), src_p, dst_p)
    out = _t4(agg3.reshape(2, NP // 128, 128), t3.reshape(NP // 128, 128),
              d.reshape(NP // 128, 128), b3.reshape(1, 1))
    return out.reshape(NP)[:N]
